# baseline (speedup 1.0000x reference)
import dataclasses
import functools

import jax
import jax.numpy as jnp
from jax import lax
from jax.experimental import pallas as pl
from jax.experimental.pallas import tpu as pltpu
from jax.experimental.pallas import tpu_sc as plsc

NEG = -100000000.0
EPS = 1.1920929e-07
ONE_M_EPS = 1.0 - EPS
BIGI = 2**30
LN2 = 0.6931471805599453

NCHUNK = 4

NC = 2
NS = 16
NW = NC * NS
L = 16

C0 = 3.1930857718625794e-05
C1 = 1.4412670742163987
C2 = -0.7057026209301708
C3 = 0.4087189439212958
C4 = -0.18772049275789568
C5 = 0.04342836333161691


def _vm_chunk(vml, vmm_f):
    x = vml * (1.0 - vmm_f) + NEG * vmm_f
    m = jnp.max(x, axis=1, keepdims=True)
    e = jnp.exp(x - m)
    s = jnp.sum(e, axis=1, keepdims=True)
    sxe = jnp.sum(e * x, axis=1, keepdims=True)
    logs = jnp.log(s)
    lse = m + logs
    vm_lp = -logs[:, 0]
    vm_ent = jnp.where(m[:, 0] == NEG, 0.0, lse[:, 0] - sxe[:, 0] / s[:, 0])
    ii = lax.broadcasted_iota(jnp.int32, x.shape, 1)
    sel_vm = jnp.min(jnp.where(x == m, ii, BIGI), axis=1)
    return sel_vm, vm_lp, vm_ent


def _vm_kernel(vml_hbm, vmm_hbm, selvm_ref, lp_ref, ent_ref,
               xb0, xb1, mb0, mb1, sems):
    xb = (xb0, xb1)
    mb = (mb0, mb1)
    cr = xb0.shape[0]

    def start(c):
        buf = c % 2
        sl = pl.ds(c * cr, cr)
        cps = (
            pltpu.make_async_copy(vml_hbm.at[sl, :], xb[buf], sems.at[buf, 0]),
            pltpu.make_async_copy(vmm_hbm.at[sl, :], mb[buf], sems.at[buf, 1]),
        )
        for cp in cps:
            cp.start()
        return cps

    pending = start(0)
    for c in range(NCHUNK):
        for cp in pending:
            cp.wait()
        if c + 1 < NCHUNK:
            pending = start(c + 1)
        buf = c % 2
        sel_vm, vm_lp, vm_ent = _vm_chunk(
            xb[buf][...], mb[buf][...].astype(jnp.float32))
        sl = pl.ds(c * cr, cr)
        selvm_ref[sl] = sel_vm
        lp_ref[sl] = vm_lp
        ent_ref[sl] = vm_ent


def _softlog16(v):
    bits = lax.bitcast_convert_type(v, jnp.int32)
    ex = (bits >> 23) - 127
    exf = ex.astype(jnp.float32)
    mant = lax.bitcast_convert_type(
        (bits & 0x007FFFFF) | 0x3F800000, jnp.float32)
    t = mant - 1.0
    p = C5
    for c in (C4, C3, C2, C1, C0):
        p = p * t + c
    return (exf + p) * LN2


def _sc_row(pbuf, mbuf, nvec):
    zero = jnp.zeros((L,), jnp.float32)

    def pass_a(j, carry):
        sacc, cacc = carry
        mv = mbuf[pl.ds(j * L, L)]
        pv = pbuf[pl.ds(j * L, L)]
        un = 1.0 - mv
        return sacc + pv * un, cacc + un

    sacc, cacc = lax.fori_loop(0, nvec, pass_a, (zero, zero))
    S = jnp.sum(sacc)
    cnt = jnp.sum(cacc)
    small = S < 0.0001
    S2 = jnp.where(small, cnt, S)
    s2v = jnp.full((L,), S2)
    smallv = jnp.full((L,), jnp.where(small, 1.0, 0.0)) > 0.5
    iota = lax.iota(jnp.int32, L)

    def pass_b(j, carry):
        eacc, rmax, ridx = carry
        mv = mbuf[pl.ds(j * L, L)]
        pv = pbuf[pl.ds(j * L, L)]
        un = 1.0 - mv
        p2 = jnp.where(smallv, un, pv * un)
        q = p2 / s2v
        qc = jnp.minimum(jnp.maximum(q, EPS), ONE_M_EPS)
        lq = _softlog16(qc)
        eacc = eacc - q * lq
        gt = q > rmax
        rmax = jnp.where(gt, q, rmax)
        ridx = jnp.where(gt, iota + j * L, ridx)
        return eacc, rmax, ridx

    eacc, rmax, ridx = lax.fori_loop(
        0, nvec, pass_b,
        (zero, jnp.full((L,), -1.0), jnp.zeros((L,), jnp.int32)))

    ent = jnp.sum(eacc)
    M = jnp.max(rmax)
    sel = jnp.min(jnp.where(rmax == jnp.full((L,), M), ridx, BIGI))
    mc = jnp.minimum(jnp.maximum(jnp.full((L,), M), EPS), ONE_M_EPS)
    lp_bits = jnp.max(lax.bitcast_convert_type(_softlog16(mc), jnp.int32))
    ent_bits = jnp.max(
        lax.bitcast_convert_type(jnp.full((L,), ent), jnp.int32))
    return sel, lp_bits, ent_bits


def _make_sc_pm(B, NP):
    rpw = B // NW
    mesh = plsc.VectorSubcoreMesh(core_axis_name="c", subcore_axis_name="s")
    cp = pltpu.CompilerParams()
    if "needs_layout_passes" in pltpu.CompilerParams.__dataclass_fields__:
        cp = dataclasses.replace(cp, needs_layout_passes=False)

    @functools.partial(
        pl.kernel, mesh=mesh, compiler_params=cp,
        out_type=jax.ShapeDtypeStruct((NW, L), jnp.int32),
        scratch_types=[
            pltpu.VMEM((NP,), jnp.float32),
            pltpu.VMEM((NP,), jnp.float32),
            pltpu.VMEM((L,), jnp.int32),
            pltpu.SemaphoreType.DMA,
        ],
    )
    def sc_pm(pp_hbm, mm_hbm, out_hbm, pbuf, mbuf, obuf, sem):
        wid = lax.axis_index("s") * NC + lax.axis_index("c")
        iota = lax.iota(jnp.int32, L)
        ovec = jnp.zeros((L,), jnp.int32)
        for r in range(rpw):
            row = wid * rpw + r
            pltpu.async_copy(pp_hbm.at[row], pbuf, sem).wait()
            pltpu.async_copy(mm_hbm.at[row], mbuf, sem).wait()
            sel, lp_bits, ent_bits = _sc_row(pbuf, mbuf, NP // L)
            for k, val in ((0, sel), (1, lp_bits), (2, ent_bits)):
                ovec = jnp.where(iota == 3 * r + k, jnp.full((L,), val), ovec)
        obuf[...] = ovec
        pltpu.sync_copy(obuf, out_hbm.at[wid])

    return sc_pm


def kernel(vm_logits, vm_mask, pm_probs, pm_mask):
    B = vm_logits.shape[0]
    NV = vm_logits.shape[1]
    NP = pm_probs.shape[1]
    CR = B // NCHUNK
    rpw = B // NW

    sel_vm, vm_lp, vm_ent = pl.pallas_call(
        _vm_kernel,
        in_specs=[pl.BlockSpec(memory_space=pl.ANY)] * 2,
        out_shape=(
            jax.ShapeDtypeStruct((B,), jnp.int32),
            jax.ShapeDtypeStruct((B,), jnp.float32),
            jax.ShapeDtypeStruct((B,), jnp.float32),
        ),
        scratch_shapes=[
            pltpu.VMEM((CR, NV), jnp.float32),
            pltpu.VMEM((CR, NV), jnp.float32),
            pltpu.VMEM((CR, NV), jnp.int8),
            pltpu.VMEM((CR, NV), jnp.int8),
            pltpu.SemaphoreType.DMA((2, 2)),
        ],
    )(vm_logits, vm_mask.view(jnp.int8))

    sc_out = _make_sc_pm(B, NP)(pm_probs, pm_mask.astype(jnp.float32))
    sc = sc_out[:, :3 * rpw].reshape(NW, rpw, 3)
    sel_pm = sc[:, :, 0].reshape(B)
    pm_lp = lax.bitcast_convert_type(sc[:, :, 1], jnp.float32).reshape(B)
    pm_ent = lax.bitcast_convert_type(sc[:, :, 2], jnp.float32).reshape(B)
    return (sel_vm, sel_pm, vm_lp + pm_lp, vm_ent + pm_ent)


# speedup vs baseline: 1.0849x; 1.0849x over previous
import dataclasses
import functools

import jax
import jax.numpy as jnp
from jax import lax
from jax.experimental import pallas as pl
from jax.experimental.pallas import tpu as pltpu
from jax.experimental.pallas import tpu_sc as plsc

NEG = -100000000.0
EPS = 1.1920929e-07
ONE_M_EPS = 1.0 - EPS
BIGI = 2**30
LN2 = 0.6931471805599453

NCHUNK = 4

NC = 2
NS = 16
NW = NC * NS
L = 16

C0 = 3.1930857718625794e-05
C1 = 1.4412670742163987
C2 = -0.7057026209301708
C3 = 0.4087189439212958
C4 = -0.18772049275789568
C5 = 0.04342836333161691


def _vm_chunk(vml, vmm_f):
    x = vml * (1.0 - vmm_f) + NEG * vmm_f
    m = jnp.max(x, axis=1, keepdims=True)
    e = jnp.exp(x - m)
    s = jnp.sum(e, axis=1, keepdims=True)
    sxe = jnp.sum(e * x, axis=1, keepdims=True)
    logs = jnp.log(s)
    lse = m + logs
    vm_lp = -logs[:, 0]
    vm_ent = jnp.where(m[:, 0] == NEG, 0.0, lse[:, 0] - sxe[:, 0] / s[:, 0])
    ii = lax.broadcasted_iota(jnp.int32, x.shape, 1)
    sel_vm = jnp.min(jnp.where(x == m, ii, BIGI), axis=1)
    return sel_vm, vm_lp, vm_ent


def _vm_kernel(vml_hbm, vmm_hbm, selvm_ref, lp_ref, ent_ref,
               xb0, xb1, mb0, mb1, sems):
    xb = (xb0, xb1)
    mb = (mb0, mb1)
    cr = xb0.shape[0]

    def start(c):
        buf = c % 2
        sl = pl.ds(c * cr, cr)
        cps = (
            pltpu.make_async_copy(vml_hbm.at[sl, :], xb[buf], sems.at[buf, 0]),
            pltpu.make_async_copy(vmm_hbm.at[sl, :], mb[buf], sems.at[buf, 1]),
        )
        for cp in cps:
            cp.start()
        return cps

    pending = start(0)
    for c in range(NCHUNK):
        for cp in pending:
            cp.wait()
        if c + 1 < NCHUNK:
            pending = start(c + 1)
        buf = c % 2
        sel_vm, vm_lp, vm_ent = _vm_chunk(
            xb[buf][...], mb[buf][...].astype(jnp.float32))
        sl = pl.ds(c * cr, cr)
        selvm_ref[sl] = sel_vm
        lp_ref[sl] = vm_lp
        ent_ref[sl] = vm_ent


def _softlog16(v):
    bits = lax.bitcast_convert_type(v, jnp.int32)
    ex = (bits >> 23) - 127
    exf = ex.astype(jnp.float32)
    mant = lax.bitcast_convert_type(
        (bits & 0x007FFFFF) | 0x3F800000, jnp.float32)
    t = mant - 1.0
    p = C5
    for c in (C4, C3, C2, C1, C0):
        p = p * t + c
    return (exf + p) * LN2


def _sc_row(pbuf, mbuf, nvec):
    zero = jnp.zeros((L,), jnp.float32)

    def pass_a(j, carry):
        sacc, cacc = carry
        mv = mbuf[pl.ds(j * L, L)]
        pv = pbuf[pl.ds(j * L, L)]
        un = 1.0 - mv
        return sacc + pv * un, cacc + un

    sacc, cacc = lax.fori_loop(0, nvec, pass_a, (zero, zero), unroll=8)
    S = jnp.sum(sacc)
    cnt = jnp.sum(cacc)
    small = S < 0.0001
    S2 = jnp.where(small, cnt, S)
    s2v = jnp.full((L,), S2)
    smallv = jnp.full((L,), jnp.where(small, 1.0, 0.0)) > 0.5
    iota = lax.iota(jnp.int32, L)

    def pass_b(j, carry):
        eacc, rmax, ridx = carry
        mv = mbuf[pl.ds(j * L, L)]
        pv = pbuf[pl.ds(j * L, L)]
        un = 1.0 - mv
        p2 = jnp.where(smallv, un, pv * un)
        q = p2 / s2v
        qc = jnp.minimum(jnp.maximum(q, EPS), ONE_M_EPS)
        lq = _softlog16(qc)
        eacc = eacc - q * lq
        gt = q > rmax
        rmax = jnp.where(gt, q, rmax)
        ridx = jnp.where(gt, iota + j * L, ridx)
        return eacc, rmax, ridx

    eacc, rmax, ridx = lax.fori_loop(
        0, nvec, pass_b,
        (zero, jnp.full((L,), -1.0), jnp.zeros((L,), jnp.int32)), unroll=8)

    ent = jnp.sum(eacc)
    M = jnp.max(rmax)
    sel = jnp.min(jnp.where(rmax == jnp.full((L,), M), ridx, BIGI))
    mc = jnp.minimum(jnp.maximum(jnp.full((L,), M), EPS), ONE_M_EPS)
    lp_bits = jnp.max(lax.bitcast_convert_type(_softlog16(mc), jnp.int32))
    ent_bits = jnp.max(
        lax.bitcast_convert_type(jnp.full((L,), ent), jnp.int32))
    return sel, lp_bits, ent_bits


def _make_sc_pm(B, NP):
    rpw = B // NW
    mesh = plsc.VectorSubcoreMesh(core_axis_name="c", subcore_axis_name="s")
    cp = pltpu.CompilerParams()
    if "needs_layout_passes" in pltpu.CompilerParams.__dataclass_fields__:
        cp = dataclasses.replace(cp, needs_layout_passes=False)

    @functools.partial(
        pl.kernel, mesh=mesh, compiler_params=cp,
        out_type=jax.ShapeDtypeStruct((NW, L), jnp.int32),
        scratch_types=[
            pltpu.VMEM((NP,), jnp.float32),
            pltpu.VMEM((NP,), jnp.float32),
            pltpu.VMEM((L,), jnp.int32),
            pltpu.SemaphoreType.DMA,
            pltpu.SemaphoreType.DMA,
        ],
    )
    def sc_pm(pp_hbm, mm_hbm, out_hbm, pbuf, mbuf, obuf, sem, sem2):
        wid = lax.axis_index("s") * NC + lax.axis_index("c")
        iota = lax.iota(jnp.int32, L)
        ovec = jnp.zeros((L,), jnp.int32)
        for r in range(rpw):
            row = wid * rpw + r
            cp1 = pltpu.async_copy(pp_hbm.at[row], pbuf, sem)
            cp2 = pltpu.async_copy(mm_hbm.at[row], mbuf, sem2)
            cp1.wait()
            cp2.wait()
            sel, lp_bits, ent_bits = _sc_row(pbuf, mbuf, NP // L)
            for k, val in ((0, sel), (1, lp_bits), (2, ent_bits)):
                ovec = jnp.where(iota == 3 * r + k, jnp.full((L,), val), ovec)
        obuf[...] = ovec
        pltpu.sync_copy(obuf, out_hbm.at[wid])

    return sc_pm


def kernel(vm_logits, vm_mask, pm_probs, pm_mask):
    B = vm_logits.shape[0]
    NV = vm_logits.shape[1]
    NP = pm_probs.shape[1]
    CR = B // NCHUNK
    rpw = B // NW

    sel_vm, vm_lp, vm_ent = pl.pallas_call(
        _vm_kernel,
        in_specs=[pl.BlockSpec(memory_space=pl.ANY)] * 2,
        out_shape=(
            jax.ShapeDtypeStruct((B,), jnp.int32),
            jax.ShapeDtypeStruct((B,), jnp.float32),
            jax.ShapeDtypeStruct((B,), jnp.float32),
        ),
        scratch_shapes=[
            pltpu.VMEM((CR, NV), jnp.float32),
            pltpu.VMEM((CR, NV), jnp.float32),
            pltpu.VMEM((CR, NV), jnp.int8),
            pltpu.VMEM((CR, NV), jnp.int8),
            pltpu.SemaphoreType.DMA((2, 2)),
        ],
    )(vm_logits, vm_mask.view(jnp.int8))

    sc_out = _make_sc_pm(B, NP)(pm_probs, pm_mask.astype(jnp.float32))
    sc = sc_out[:, :3 * rpw].reshape(NW, rpw, 3)
    sel_pm = sc[:, :, 0].reshape(B)
    pm_lp = lax.bitcast_convert_type(sc[:, :, 1], jnp.float32).reshape(B)
    pm_ent = lax.bitcast_convert_type(sc[:, :, 2], jnp.float32).reshape(B)
    return (sel_vm, sel_pm, vm_lp + pm_lp, vm_ent + pm_ent)


# speedup vs baseline: 2.8713x; 2.6466x over previous
import jax
import jax.numpy as jnp
from jax.experimental import pallas as pl
from jax.experimental.pallas import tpu as pltpu

NEG = -100000000.0
EPS = 1.1920929e-07
BIGI = 2**30

NCHUNK = 4


def _chunk_compute(vml, vmm_f, pp, un):
    x = vml * (1.0 - vmm_f) + NEG * vmm_f
    m = jnp.max(x, axis=1, keepdims=True)
    e = jnp.exp(x - m)
    s = jnp.sum(e, axis=1, keepdims=True)
    sxe = jnp.sum(e * x, axis=1, keepdims=True)
    logs = jnp.log(s)
    lse = m + logs
    vm_lp = -logs[:, 0]
    vm_ent = jnp.where(m[:, 0] == NEG, 0.0, lse[:, 0] - sxe[:, 0] / s[:, 0])
    ii = jax.lax.broadcasted_iota(jnp.int32, x.shape, 1)
    sel_vm = jnp.min(jnp.where(x == m, ii, BIGI), axis=1)

    p = pp * un
    S = jnp.sum(p, axis=1, keepdims=True)
    cnt = jnp.sum(un, axis=1, keepdims=True)
    small = S < 0.0001
    p2 = jnp.where(small, un, p)
    S2 = jnp.where(small, cnt, S)
    q = p2 / S2
    lq = jnp.log(jnp.clip(q, EPS, 1.0 - EPS))
    pm_ent = -jnp.sum(lq * q, axis=1)
    mq = jnp.max(q, axis=1, keepdims=True)
    jj = jax.lax.broadcasted_iota(jnp.int32, q.shape, 1)
    sel_pm = jnp.min(jnp.where(q == mq, jj, BIGI), axis=1)
    pm_lp = jnp.log(jnp.clip(mq[:, 0], EPS, 1.0 - EPS))
    return sel_vm, sel_pm, vm_lp + pm_lp, vm_ent + pm_ent


def _heads_kernel(vml_hbm, vmm_hbm, pmp_hbm, pmm_hbm,
                  selvm_ref, selpm_ref, lp_ref, ent_ref,
                  xb0, xb1, mb0, mb1, pb0, pb1, qb0, qb1, sems):
    xb = (xb0, xb1)
    mb = (mb0, mb1)
    pb = (pb0, pb1)
    qb = (qb0, qb1)
    cr = xb0.shape[0]

    def start(c):
        buf = c % 2
        sl = pl.ds(c * cr, cr)
        cps = (
            pltpu.make_async_copy(vml_hbm.at[sl, :], xb[buf], sems.at[buf, 0]),
            pltpu.make_async_copy(vmm_hbm.at[sl, :], mb[buf], sems.at[buf, 1]),
            pltpu.make_async_copy(pmp_hbm.at[sl, :], pb[buf], sems.at[buf, 2]),
            pltpu.make_async_copy(pmm_hbm.at[sl, :], qb[buf], sems.at[buf, 3]),
        )
        for cp in cps:
            cp.start()
        return cps

    pending = start(0)
    for c in range(NCHUNK):
        for cp in pending:
            cp.wait()
        if c + 1 < NCHUNK:
            pending = start(c + 1)
        buf = c % 2
        sel_vm, sel_pm, lp, ent = _chunk_compute(
            xb[buf][...],
            mb[buf][...].astype(jnp.float32),
            pb[buf][...],
            1.0 - qb[buf][...].astype(jnp.float32),
        )
        sl = pl.ds(c * cr, cr)
        selvm_ref[sl] = sel_vm
        selpm_ref[sl] = sel_pm
        lp_ref[sl] = lp
        ent_ref[sl] = ent


def kernel(vm_logits, vm_mask, pm_probs, pm_mask):
    B = vm_logits.shape[0]
    NV = vm_logits.shape[1]
    NP = pm_probs.shape[1]
    CR = B // NCHUNK
    out = pl.pallas_call(
        _heads_kernel,
        in_specs=[pl.BlockSpec(memory_space=pl.ANY)] * 4,
        out_shape=(
            jax.ShapeDtypeStruct((B,), jnp.int32),
            jax.ShapeDtypeStruct((B,), jnp.int32),
            jax.ShapeDtypeStruct((B,), jnp.float32),
            jax.ShapeDtypeStruct((B,), jnp.float32),
        ),
        scratch_shapes=[
            pltpu.VMEM((CR, NV), jnp.float32),
            pltpu.VMEM((CR, NV), jnp.float32),
            pltpu.VMEM((CR, NV), jnp.int8),
            pltpu.VMEM((CR, NV), jnp.int8),
            pltpu.VMEM((CR, NP), jnp.float32),
            pltpu.VMEM((CR, NP), jnp.float32),
            pltpu.VMEM((CR, NP), jnp.int8),
            pltpu.VMEM((CR, NP), jnp.int8),
            pltpu.SemaphoreType.DMA((2, 4)),
        ],
    )(vm_logits, vm_mask.view(jnp.int8), pm_probs, pm_mask.view(jnp.int8))
    return out


# speedup vs baseline: 2.9012x; 1.0104x over previous
import jax
import jax.numpy as jnp
from jax.experimental import pallas as pl
from jax.experimental.pallas import tpu as pltpu

NEG = -100000000.0
EPS = 1.1920929e-07
BIGI = 2**30

NCHUNK = 4


def _chunk_compute(vml, vmm_f, pp, un):
    x = jnp.where(vmm_f != 0.0, NEG, vml)
    m = jnp.max(x, axis=1, keepdims=True)
    e = jnp.exp(x - m)
    s = jnp.sum(e, axis=1, keepdims=True)
    sxe = jnp.sum(e * x, axis=1, keepdims=True)
    logs = jnp.log(s)
    lse = m + logs
    vm_lp = -logs[:, 0]
    vm_ent = jnp.where(m[:, 0] == NEG, 0.0, lse[:, 0] - sxe[:, 0] / s[:, 0])
    ii = jax.lax.broadcasted_iota(jnp.int32, x.shape, 1)
    sel_vm = jnp.min(jnp.where(x == m, ii, BIGI), axis=1)

    p = pp * un
    S = jnp.sum(p, axis=1, keepdims=True)
    cnt = jnp.sum(un, axis=1, keepdims=True)
    small = S < 0.0001
    p2 = jnp.where(small, un, p)
    S2 = jnp.where(small, cnt, S)
    q = p2 / S2
    lq = jnp.log(jnp.clip(q, EPS, 1.0 - EPS))
    pm_ent = -jnp.sum(lq * q, axis=1)
    mq = jnp.max(q, axis=1, keepdims=True)
    jj = jax.lax.broadcasted_iota(jnp.int32, q.shape, 1)
    sel_pm = jnp.min(jnp.where(q == mq, jj, BIGI), axis=1)
    pm_lp = jnp.log(jnp.clip(mq[:, 0], EPS, 1.0 - EPS))
    return sel_vm, sel_pm, vm_lp + pm_lp, vm_ent + pm_ent


def _heads_kernel(vml_hbm, vmm_hbm, pmp_hbm, pmm_hbm,
                  selvm_ref, selpm_ref, lp_ref, ent_ref,
                  xb0, xb1, mb0, mb1, pb0, pb1, qb0, qb1, sems):
    xb = (xb0, xb1)
    mb = (mb0, mb1)
    pb = (pb0, pb1)
    qb = (qb0, qb1)
    cr = xb0.shape[0]

    def start(c):
        buf = c % 2
        sl = pl.ds(c * cr, cr)
        cps = (
            pltpu.make_async_copy(vml_hbm.at[sl, :], xb[buf], sems.at[buf, 0]),
            pltpu.make_async_copy(vmm_hbm.at[sl, :], mb[buf], sems.at[buf, 1]),
            pltpu.make_async_copy(pmp_hbm.at[sl, :], pb[buf], sems.at[buf, 2]),
            pltpu.make_async_copy(pmm_hbm.at[sl, :], qb[buf], sems.at[buf, 3]),
        )
        for cp in cps:
            cp.start()
        return cps

    pending = start(0)
    for c in range(NCHUNK):
        for cp in pending:
            cp.wait()
        if c + 1 < NCHUNK:
            pending = start(c + 1)
        buf = c % 2
        sel_vm, sel_pm, lp, ent = _chunk_compute(
            xb[buf][...],
            mb[buf][...].astype(jnp.float32),
            pb[buf][...],
            1.0 - qb[buf][...].astype(jnp.float32),
        )
        sl = pl.ds(c * cr, cr)
        selvm_ref[sl] = sel_vm
        selpm_ref[sl] = sel_pm
        lp_ref[sl] = lp
        ent_ref[sl] = ent


def kernel(vm_logits, vm_mask, pm_probs, pm_mask):
    B = vm_logits.shape[0]
    NV = vm_logits.shape[1]
    NP = pm_probs.shape[1]
    CR = B // NCHUNK
    out = pl.pallas_call(
        _heads_kernel,
        in_specs=[pl.BlockSpec(memory_space=pl.ANY)] * 4,
        out_shape=(
            jax.ShapeDtypeStruct((B,), jnp.int32),
            jax.ShapeDtypeStruct((B,), jnp.int32),
            jax.ShapeDtypeStruct((B,), jnp.float32),
            jax.ShapeDtypeStruct((B,), jnp.float32),
        ),
        scratch_shapes=[
            pltpu.VMEM((CR, NV), jnp.float32),
            pltpu.VMEM((CR, NV), jnp.float32),
            pltpu.VMEM((CR, NV), jnp.int8),
            pltpu.VMEM((CR, NV), jnp.int8),
            pltpu.VMEM((CR, NP), jnp.float32),
            pltpu.VMEM((CR, NP), jnp.float32),
            pltpu.VMEM((CR, NP), jnp.int8),
            pltpu.VMEM((CR, NP), jnp.int8),
            pltpu.SemaphoreType.DMA((2, 4)),
        ],
    )(vm_logits, vm_mask.view(jnp.int8), pm_probs, pm_mask.view(jnp.int8))
    return out


# speedup vs baseline: 3.1768x; 1.0950x over previous
import jax
import jax.numpy as jnp
from jax.experimental import pallas as pl
from jax.experimental.pallas import tpu as pltpu

NEG = -100000000.0
EPS = 1.1920929e-07
BIGI = 2**30

NCHUNK = 2


def _chunk_compute(vml, vmm_f, pp, un):
    x = jnp.where(vmm_f != 0.0, NEG, vml)
    m = jnp.max(x, axis=1, keepdims=True)
    e = jnp.exp(x - m)
    s = jnp.sum(e, axis=1, keepdims=True)
    sxe = jnp.sum(e * x, axis=1, keepdims=True)
    logs = jnp.log(s)
    lse = m + logs
    vm_lp = -logs[:, 0]
    vm_ent = jnp.where(m[:, 0] == NEG, 0.0, lse[:, 0] - sxe[:, 0] / s[:, 0])
    ii = jax.lax.broadcasted_iota(jnp.int32, x.shape, 1)
    sel_vm = jnp.min(jnp.where(x == m, ii, BIGI), axis=1)

    p = pp * un
    S = jnp.sum(p, axis=1, keepdims=True)
    cnt = jnp.sum(un, axis=1, keepdims=True)
    small = S < 0.0001
    p2 = jnp.where(small, un, p)
    S2 = jnp.where(small, cnt, S)
    q = p2 / S2
    lq = jnp.log(jnp.clip(q, EPS, 1.0 - EPS))
    pm_ent = -jnp.sum(lq * q, axis=1)
    mq = jnp.max(q, axis=1, keepdims=True)
    jj = jax.lax.broadcasted_iota(jnp.int32, q.shape, 1)
    sel_pm = jnp.min(jnp.where(q == mq, jj, BIGI), axis=1)
    pm_lp = jnp.log(jnp.clip(mq[:, 0], EPS, 1.0 - EPS))
    return sel_vm, sel_pm, vm_lp + pm_lp, vm_ent + pm_ent


def _heads_kernel(vml_hbm, vmm_hbm, pmp_hbm, pmm_hbm,
                  selvm_ref, selpm_ref, lp_ref, ent_ref,
                  xb0, xb1, mb0, mb1, pb0, pb1, qb0, qb1, sems):
    xb = (xb0, xb1)
    mb = (mb0, mb1)
    pb = (pb0, pb1)
    qb = (qb0, qb1)
    cr = xb0.shape[0]

    def start(c):
        buf = c % 2
        sl = pl.ds(c * cr, cr)
        cps = (
            pltpu.make_async_copy(vml_hbm.at[sl, :], xb[buf], sems.at[buf, 0]),
            pltpu.make_async_copy(vmm_hbm.at[sl, :], mb[buf], sems.at[buf, 1]),
            pltpu.make_async_copy(pmp_hbm.at[sl, :], pb[buf], sems.at[buf, 2]),
            pltpu.make_async_copy(pmm_hbm.at[sl, :], qb[buf], sems.at[buf, 3]),
        )
        for cp in cps:
            cp.start()
        return cps

    pending = start(0)
    for c in range(NCHUNK):
        for cp in pending:
            cp.wait()
        if c + 1 < NCHUNK:
            pending = start(c + 1)
        buf = c % 2
        sel_vm, sel_pm, lp, ent = _chunk_compute(
            xb[buf][...],
            mb[buf][...].astype(jnp.float32),
            pb[buf][...],
            1.0 - qb[buf][...].astype(jnp.float32),
        )
        sl = pl.ds(c * cr, cr)
        selvm_ref[sl] = sel_vm
        selpm_ref[sl] = sel_pm
        lp_ref[sl] = lp
        ent_ref[sl] = ent


def kernel(vm_logits, vm_mask, pm_probs, pm_mask):
    B = vm_logits.shape[0]
    NV = vm_logits.shape[1]
    NP = pm_probs.shape[1]
    CR = B // NCHUNK
    out = pl.pallas_call(
        _heads_kernel,
        in_specs=[pl.BlockSpec(memory_space=pl.ANY)] * 4,
        out_shape=(
            jax.ShapeDtypeStruct((B,), jnp.int32),
            jax.ShapeDtypeStruct((B,), jnp.int32),
            jax.ShapeDtypeStruct((B,), jnp.float32),
            jax.ShapeDtypeStruct((B,), jnp.float32),
        ),
        scratch_shapes=[
            pltpu.VMEM((CR, NV), jnp.float32),
            pltpu.VMEM((CR, NV), jnp.float32),
            pltpu.VMEM((CR, NV), jnp.int8),
            pltpu.VMEM((CR, NV), jnp.int8),
            pltpu.VMEM((CR, NP), jnp.float32),
            pltpu.VMEM((CR, NP), jnp.float32),
            pltpu.VMEM((CR, NP), jnp.int8),
            pltpu.VMEM((CR, NP), jnp.int8),
            pltpu.SemaphoreType.DMA((2, 4)),
        ],
    )(vm_logits, vm_mask.view(jnp.int8), pm_probs, pm_mask.view(jnp.int8))
    return out
